# Initial kernel scaffold; baseline (speedup 1.0000x reference)
import jax
import jax.numpy as jnp
from jax.experimental import pallas as pl


def kernel(x, edge_index, hyperedge_index, W1, b1, W2, att_src2, att_dst2, b2, W3, att_src3, att_dst3, b3):
    raise NotImplementedError("write your pallas kernel here")


# speedup vs baseline: 23.5616x; 23.5616x over previous
import functools

import jax
import jax.numpy as jnp
from jax import lax
from jax.experimental import pallas as pl
from jax.experimental.pallas import tpu as pltpu
from jax.experimental.pallas import tpu_sc as plsc

N = 10000
E = 320000
NNZ = 160000
D_IN = 128
D_H = 128
D_OUT = 64

NC = 2
NS = 16
NW = NC * NS
STRIPE = 632
TAIL = N - 15 * STRIPE
CHUNK = 200

f32 = jnp.float32
i32 = jnp.int32


def _sc_mesh():
    return plsc.VectorSubcoreMesh(
        core_axis_name="c", subcore_axis_name="s", num_cores=NC, num_subcores=NS
    )


def _degrees(nidx, eidx, ones_t, zvec):
    per_t = NNZ // NW

    @functools.partial(
        pl.kernel,
        out_type=(
            jax.ShapeDtypeStruct((NC, N), f32),
            jax.ShapeDtypeStruct((NC, N), f32),
        ),
        mesh=_sc_mesh(),
        scratch_types=[
            pltpu.VMEM((per_t,), i32),
            pltpu.VMEM((per_t,), f32),
            pltpu.VMEM_SHARED((N,), f32),
            pltpu.VMEM_SHARED((N,), f32),
        ],
    )
    def k(nidx_h, eidx_h, ones_h, zvec_h, dn_h, de_h, idx_v, ones_v, dn_s, de_s):
        cid = lax.axis_index("c")
        sid = lax.axis_index("s")
        wid = cid * NS + sid
        base = wid * per_t

        @pl.when(sid == 0)
        def _():
            pltpu.sync_copy(zvec_h, dn_s)
            pltpu.sync_copy(zvec_h, de_s)

        pltpu.sync_copy(ones_h, ones_v)
        pltpu.sync_copy(nidx_h.at[pl.ds(base, per_t)], idx_v)
        plsc.subcore_barrier()
        pltpu.sync_copy(ones_v, dn_s.at[idx_v], add=True)
        pltpu.sync_copy(eidx_h.at[pl.ds(base, per_t)], idx_v)
        pltpu.sync_copy(ones_v, de_s.at[idx_v], add=True)
        plsc.subcore_barrier()

        @pl.when(sid == 0)
        def _():
            pltpu.sync_copy(dn_s, dn_h.at[cid])
            pltpu.sync_copy(de_s, de_h.at[cid])

    return k(nidx, eidx, ones_t, zvec)


def _hop(table, gidx, sidx, coef, zeros2d, d, n_edges):
    per_t = n_edges // NW
    nch = per_t // CHUNK
    with_coef = coef is not None

    scratch = [
        pltpu.VMEM((CHUNK,), i32),
        pltpu.VMEM((CHUNK,), i32),
        pltpu.VMEM((CHUNK, d), f32),
        pltpu.VMEM((CHUNK,), f32),
        pltpu.VMEM_SHARED((N, d), f32),
        pltpu.SemaphoreType.DMA,
    ]

    def body(table_h, gidx_h, sidx_h, coef_h, z2_h, out_h,
             gi_v, si_v, rows_v, cf_v, acc_s, sem):
        cid = lax.axis_index("c")
        sid = lax.axis_index("s")
        wid = cid * NS + sid
        base = wid * per_t
        r0 = pl.multiple_of(sid * STRIPE, 8)

        @pl.when(sid < 15)
        def _():
            pltpu.sync_copy(z2_h.at[pl.ds(r0, STRIPE)], acc_s.at[pl.ds(r0, STRIPE)])

        @pl.when(sid == 15)
        def _():
            pltpu.sync_copy(z2_h.at[pl.ds(15 * STRIPE, TAIL)],
                            acc_s.at[pl.ds(15 * STRIPE, TAIL)])

        plsc.subcore_barrier()

        def chunk(kk, _):
            off = base + kk * CHUNK
            pltpu.sync_copy(gidx_h.at[pl.ds(off, CHUNK)], gi_v)
            pltpu.sync_copy(sidx_h.at[pl.ds(off, CHUNK)], si_v)
            pltpu.async_copy(table_h.at[gi_v], rows_v, sem).wait()
            if with_coef:
                pltpu.sync_copy(coef_h.at[pl.ds(off, CHUNK)], cf_v)

                def row(i, _):
                    cb = plsc.load_gather(cf_v, [jnp.full((16,), i, i32)])
                    for m in range(d // 16):
                        sl = (i, pl.ds(16 * m, 16))
                        rows_v[sl] = rows_v[sl] * cb
                    return 0

                lax.fori_loop(0, CHUNK, row, 0)
            pltpu.sync_copy(rows_v, acc_s.at[si_v], add=True)
            return 0

        lax.fori_loop(0, nch, chunk, 0)
        plsc.subcore_barrier()

        @pl.when(sid < 15)
        def _():
            pltpu.sync_copy(acc_s.at[pl.ds(r0, STRIPE)],
                            out_h.at[cid, pl.ds(r0, STRIPE)])

        @pl.when(sid == 15)
        def _():
            pltpu.sync_copy(acc_s.at[pl.ds(15 * STRIPE, TAIL)],
                            out_h.at[cid, pl.ds(15 * STRIPE, TAIL)])

    if with_coef:
        def body_c(table_h, gidx_h, sidx_h, coef_h, z2_h, out_h,
                   gi_v, si_v, rows_v, cf_v, acc_s, sem):
            body(table_h, gidx_h, sidx_h, coef_h, z2_h, out_h,
                 gi_v, si_v, rows_v, cf_v, acc_s, sem)

        k = functools.partial(
            pl.kernel,
            out_type=jax.ShapeDtypeStruct((NC, N, d), f32),
            mesh=_sc_mesh(),
            compiler_params=pltpu.CompilerParams(needs_layout_passes=False),
            scratch_types=scratch,
        )(body_c)
        return k(table, gidx, sidx, coef, zeros2d)

    def body_n(table_h, gidx_h, sidx_h, z2_h, out_h,
               gi_v, si_v, rows_v, cf_v, acc_s, sem):
        body(table_h, gidx_h, sidx_h, None, z2_h, out_h,
             gi_v, si_v, rows_v, cf_v, acc_s, sem)

    k = functools.partial(
        pl.kernel,
        out_type=jax.ShapeDtypeStruct((NC, N, d), f32),
        mesh=_sc_mesh(),
        compiler_params=pltpu.CompilerParams(needs_layout_passes=False),
        scratch_types=scratch,
    )(body_n)
    return k(table, gidx, sidx, zeros2d)


def _gat_alpha(asrc, adst, src, dst, shift16, zvec):
    per_t = E // NW
    nv = per_t // 16

    @functools.partial(
        pl.kernel,
        out_type=(
            jax.ShapeDtypeStruct((E,), f32),
            jax.ShapeDtypeStruct((NC, N), f32),
        ),
        mesh=_sc_mesh(),
        compiler_params=pltpu.CompilerParams(needs_layout_passes=False),
        scratch_types=[
            pltpu.VMEM((N,), f32),
            pltpu.VMEM((N,), f32),
            pltpu.VMEM((per_t,), i32),
            pltpu.VMEM((per_t,), i32),
            pltpu.VMEM((per_t,), f32),
            pltpu.VMEM((16,), f32),
            pltpu.VMEM_SHARED((N,), f32),
        ],
    )
    def k(as_h, ad_h, src_h, dst_h, sh_h, zv_h, ex_h, den_h,
          as_v, ad_v, src_v, dst_v, ex_v, sh_v, den_s):
        cid = lax.axis_index("c")
        sid = lax.axis_index("s")
        wid = cid * NS + sid
        base = wid * per_t

        @pl.when(sid == 0)
        def _():
            pltpu.sync_copy(zv_h, den_s)

        pltpu.sync_copy(as_h, as_v)
        pltpu.sync_copy(ad_h, ad_v)
        pltpu.sync_copy(src_h.at[pl.ds(base, per_t)], src_v)
        pltpu.sync_copy(dst_h.at[pl.ds(base, per_t)], dst_v)
        pltpu.sync_copy(sh_h, sh_v)
        plsc.subcore_barrier()
        cvec = sh_v[...]

        def body(j, _):
            sl = pl.ds(16 * j, 16)
            a = plsc.load_gather(as_v, [src_v[sl]])
            b = plsc.load_gather(ad_v, [dst_v[sl]])
            z = a + b
            z = jnp.where(z >= 0.0, z, 0.2 * z)
            ex_v[sl] = jnp.exp(z - cvec)
            return 0

        lax.fori_loop(0, nv, body, 0)
        pltpu.sync_copy(ex_v, ex_h.at[pl.ds(base, per_t)])
        pltpu.sync_copy(ex_v, den_s.at[dst_v], add=True)
        plsc.subcore_barrier()

        @pl.when(sid == 0)
        def _():
            pltpu.sync_copy(den_s, den_h.at[cid])

    return k(asrc, adst, src, dst, shift16, zvec)


_R = 1000


def _mm1(x, w1):
    def body(x_ref, w_ref, o_ref):
        o_ref[...] = jnp.dot(x_ref[...], w_ref[...], preferred_element_type=f32)

    return pl.pallas_call(
        body,
        grid=(N // _R,),
        in_specs=[
            pl.BlockSpec((_R, D_IN), lambda i: (i, 0)),
            pl.BlockSpec((D_IN, D_H), lambda i: (0, 0)),
        ],
        out_specs=pl.BlockSpec((_R, D_H), lambda i: (i, 0)),
        out_shape=jax.ShapeDtypeStruct((N, D_H), f32),
    )(x, w1)


def _scale_e(e_p, de_p):
    def body(e_ref, d_ref, o_ref):
        de = d_ref[0] + d_ref[1]
        inv = jnp.where(de > 0.0, 1.0 / de, 0.0)
        o_ref[...] = (e_ref[0] + e_ref[1]) * inv

    return pl.pallas_call(
        body,
        grid=(N // _R,),
        in_specs=[
            pl.BlockSpec((NC, _R, D_H), lambda i: (0, i, 0)),
            pl.BlockSpec((NC, _R, 1), lambda i: (0, i, 0)),
        ],
        out_specs=pl.BlockSpec((_R, D_H), lambda i: (i, 0)),
        out_shape=jax.ShapeDtypeStruct((N, D_H), f32),
    )(e_p, de_p)


def _h1_mm2(o_p, dn_p, b1, w2, a_s, a_d):
    def body(o_ref, d_ref, b_ref, w_ref, s_ref, t_ref, xl_ref, as_ref, ad_ref):
        dn = d_ref[0] + d_ref[1]
        inv = jnp.where(dn > 0.0, 1.0 / dn, 0.0)
        h = jnp.maximum((o_ref[0] + o_ref[1]) * inv + b_ref[...], 0.0)
        xl = jnp.dot(h, w_ref[...], preferred_element_type=f32)
        xl_ref[...] = xl
        as_ref[...] = jnp.sum(xl * s_ref[...], axis=-1, keepdims=True)
        ad_ref[...] = jnp.sum(xl * t_ref[...], axis=-1, keepdims=True)

    return pl.pallas_call(
        body,
        grid=(N // _R,),
        in_specs=[
            pl.BlockSpec((NC, _R, D_H), lambda i: (0, i, 0)),
            pl.BlockSpec((NC, _R, 1), lambda i: (0, i, 0)),
            pl.BlockSpec((1, D_H), lambda i: (0, 0)),
            pl.BlockSpec((D_H, D_H), lambda i: (0, 0)),
            pl.BlockSpec((1, D_H), lambda i: (0, 0)),
            pl.BlockSpec((1, D_H), lambda i: (0, 0)),
        ],
        out_specs=[
            pl.BlockSpec((_R, D_H), lambda i: (i, 0)),
            pl.BlockSpec((_R, 1), lambda i: (i, 0)),
            pl.BlockSpec((_R, 1), lambda i: (i, 0)),
        ],
        out_shape=[
            jax.ShapeDtypeStruct((N, D_H), f32),
            jax.ShapeDtypeStruct((N, 1), f32),
            jax.ShapeDtypeStruct((N, 1), f32),
        ],
    )(o_p, dn_p, b1, w2, a_s, a_d)


def _self_terms(a_s, a_d):
    def body(s_ref, t_ref, sh_ref, ex_ref):
        m = jnp.max(s_ref[...]) + jnp.max(t_ref[...])
        c = jnp.where(m >= 0.0, m, 0.2 * m)
        sh_ref[...] = jnp.full((1, 16), c, f32)
        z = s_ref[...] + t_ref[...]
        z = jnp.where(z >= 0.0, z, 0.2 * z)
        ex_ref[...] = jnp.exp(z - c)

    return pl.pallas_call(
        body,
        in_specs=[
            pl.BlockSpec((N, 1), lambda: (0, 0)),
            pl.BlockSpec((N, 1), lambda: (0, 0)),
        ],
        out_specs=[
            pl.BlockSpec((1, 16), lambda: (0, 0)),
            pl.BlockSpec((N, 1), lambda: (0, 0)),
        ],
        out_shape=[
            jax.ShapeDtypeStruct((1, 16), f32),
            jax.ShapeDtypeStruct((N, 1), f32),
        ],
    )(a_s, a_d)


def _gat_comb(p, exs, xl, den_p, b, w_next=None, a_s=None, a_d=None, d_out=None,
              out_slice=None):
    d = xl.shape[-1]
    fused = w_next is not None
    o = out_slice if out_slice is not None else d

    def body(p_ref, e_ref, x_ref, d_ref, b_ref, *rest):
        if fused:
            w_ref, s_ref, t_ref, xl_ref, as_ref, ad_ref = rest
        else:
            (h_ref,) = rest
        den = d_ref[0] + d_ref[1] + e_ref[...]
        acc = p_ref[0] + p_ref[1] + e_ref[...] * x_ref[...]
        h = jnp.maximum(acc / (den + 1e-16) + b_ref[...], 0.0)
        if fused:
            xl = jnp.dot(h, w_ref[...], preferred_element_type=f32)
            xl_ref[...] = xl
            as_ref[...] = jnp.sum(xl * s_ref[...], axis=-1, keepdims=True)
            ad_ref[...] = jnp.sum(xl * t_ref[...], axis=-1, keepdims=True)
        else:
            h_ref[...] = h[:, :o]

    in_specs = [
        pl.BlockSpec((NC, _R, d), lambda i: (0, i, 0)),
        pl.BlockSpec((_R, 1), lambda i: (i, 0)),
        pl.BlockSpec((_R, d), lambda i: (i, 0)),
        pl.BlockSpec((NC, _R, 1), lambda i: (0, i, 0)),
        pl.BlockSpec((1, d), lambda i: (0, 0)),
    ]
    args = [p, exs, xl, den_p, b]
    if fused:
        in_specs += [
            pl.BlockSpec((d, d_out), lambda i: (0, 0)),
            pl.BlockSpec((1, d_out), lambda i: (0, 0)),
            pl.BlockSpec((1, d_out), lambda i: (0, 0)),
        ]
        args += [w_next, a_s, a_d]
        out_specs = [
            pl.BlockSpec((_R, d_out), lambda i: (i, 0)),
            pl.BlockSpec((_R, 1), lambda i: (i, 0)),
            pl.BlockSpec((_R, 1), lambda i: (i, 0)),
        ]
        out_shape = [
            jax.ShapeDtypeStruct((N, d_out), f32),
            jax.ShapeDtypeStruct((N, 1), f32),
            jax.ShapeDtypeStruct((N, 1), f32),
        ]
    else:
        out_specs = pl.BlockSpec((_R, o), lambda i: (i, 0))
        out_shape = jax.ShapeDtypeStruct((N, o), f32)

    return pl.pallas_call(
        body,
        grid=(N // _R,),
        in_specs=in_specs,
        out_specs=out_specs,
        out_shape=out_shape,
    )(*args)


def kernel(x, edge_index, hyperedge_index, W1, b1, W2, att_src2, att_dst2, b2,
           W3, att_src3, att_dst3, b3):
    nidx = hyperedge_index[0]
    eidx = hyperedge_index[1]
    src = edge_index[0]
    dst = edge_index[1]

    zvec = jnp.zeros((N,), f32)
    z2h = jnp.zeros((N, D_H), f32)
    z2o = jnp.zeros((N, D_OUT), f32)
    ones_t = jnp.ones((NNZ // NW,), f32)

    x1 = _mm1(x, W1)
    dn_p, de_p = _degrees(nidx, eidx, ones_t, zvec)
    e_p = _hop(x1, nidx, eidx, None, z2h, D_H, NNZ)
    e_s = _scale_e(e_p, de_p.reshape(NC, N, 1))
    o_p = _hop(e_s, eidx, nidx, None, z2h, D_H, NNZ)

    xl2, as2, ad2 = _h1_mm2(
        o_p, dn_p.reshape(NC, N, 1), b1.reshape(1, D_H), W2,
        att_src2.reshape(1, D_H), att_dst2.reshape(1, D_H))
    sh2, exs2 = _self_terms(as2, ad2)
    ex2, den2p = _gat_alpha(
        as2.reshape(N), ad2.reshape(N), src, dst, sh2.reshape(16), zvec)
    p2 = _hop(xl2, src, dst, ex2, z2h, D_H, E)

    pad = D_H - D_OUT
    w3p = jnp.pad(W3, ((0, 0), (0, pad)))
    xl3, as3, ad3 = _gat_comb(
        p2, exs2, xl2, den2p.reshape(NC, N, 1), b2.reshape(1, D_H),
        w_next=w3p, a_s=jnp.pad(att_src3, (0, pad)).reshape(1, D_H),
        a_d=jnp.pad(att_dst3, (0, pad)).reshape(1, D_H), d_out=D_H)
    sh3, exs3 = _self_terms(as3, ad3)
    ex3, den3p = _gat_alpha(
        as3.reshape(N), ad3.reshape(N), src, dst, sh3.reshape(16), zvec)
    p3 = _hop(xl3, src, dst, ex3, z2h, D_H, E)

    return _gat_comb(p3, exs3, xl3, den3p.reshape(NC, N, 1),
                     jnp.pad(b3, (0, pad)).reshape(1, D_H), out_slice=D_OUT)


# speedup vs baseline: 35.2010x; 1.4940x over previous
import functools

import jax
import jax.numpy as jnp
from jax import lax
from jax.experimental import pallas as pl
from jax.experimental.pallas import tpu as pltpu
from jax.experimental.pallas import tpu_sc as plsc

N = 10000
E = 320000
NNZ = 160000
D_IN = 128
D_H = 128
D_OUT = 64

NC = 2
NS = 16
NW = NC * NS
STRIPE = 632
TAIL = N - 15 * STRIPE
CHUNK = 200

f32 = jnp.float32
i32 = jnp.int32


def _sc_mesh():
    return plsc.VectorSubcoreMesh(
        core_axis_name="c", subcore_axis_name="s", num_cores=NC, num_subcores=NS
    )


def _degrees(nidx, eidx, ones_t, zvec):
    per_t = NNZ // NW

    @functools.partial(
        pl.kernel,
        out_type=(
            jax.ShapeDtypeStruct((NC, N), f32),
            jax.ShapeDtypeStruct((NC, N), f32),
        ),
        mesh=_sc_mesh(),
        scratch_types=[
            pltpu.VMEM((per_t,), i32),
            pltpu.VMEM((per_t,), f32),
            pltpu.VMEM_SHARED((N,), f32),
            pltpu.VMEM_SHARED((N,), f32),
        ],
    )
    def k(nidx_h, eidx_h, ones_h, zvec_h, dn_h, de_h, idx_v, ones_v, dn_s, de_s):
        cid = lax.axis_index("c")
        sid = lax.axis_index("s")
        wid = cid * NS + sid
        base = wid * per_t

        @pl.when(sid == 0)
        def _():
            pltpu.sync_copy(zvec_h, dn_s)
            pltpu.sync_copy(zvec_h, de_s)

        pltpu.sync_copy(ones_h, ones_v)
        pltpu.sync_copy(nidx_h.at[pl.ds(base, per_t)], idx_v)
        plsc.subcore_barrier()
        pltpu.sync_copy(ones_v, dn_s.at[idx_v], add=True)
        pltpu.sync_copy(eidx_h.at[pl.ds(base, per_t)], idx_v)
        pltpu.sync_copy(ones_v, de_s.at[idx_v], add=True)
        plsc.subcore_barrier()

        @pl.when(sid == 0)
        def _():
            pltpu.sync_copy(dn_s, dn_h.at[cid])
            pltpu.sync_copy(de_s, de_h.at[cid])

    return k(nidx, eidx, ones_t, zvec)


def _hop(table, gidx, sidx, coef, zeros2d, d, n_edges, ch):
    per_t = n_edges // NW
    nch = per_t // ch
    assert nch * ch == per_t and ch % 8 == 0
    with_coef = coef is not None

    scratch = [
        pltpu.VMEM((per_t,), i32),
        pltpu.VMEM((per_t,), i32),
        pltpu.VMEM((ch, d), f32),
        pltpu.VMEM((ch, d), f32),
        pltpu.VMEM((per_t,), f32),
        pltpu.VMEM_SHARED((N, d), f32),
        pltpu.SemaphoreType.DMA,
        pltpu.SemaphoreType.DMA,
        pltpu.SemaphoreType.DMA,
        pltpu.SemaphoreType.DMA,
    ]

    def body(table_h, gidx_h, sidx_h, coef_h, z2_h, out_h,
             gi_v, si_v, rows0, rows1, cf_v, acc_s, g0, g1, s0, s1):
        cid = lax.axis_index("c")
        sid = lax.axis_index("s")
        wid = cid * NS + sid
        base = wid * per_t
        r0 = pl.multiple_of(sid * STRIPE, 8)

        @pl.when(sid < 15)
        def _():
            pltpu.sync_copy(z2_h.at[pl.ds(r0, STRIPE)], acc_s.at[pl.ds(r0, STRIPE)])

        @pl.when(sid == 15)
        def _():
            pltpu.sync_copy(z2_h.at[pl.ds(15 * STRIPE, TAIL)],
                            acc_s.at[pl.ds(15 * STRIPE, TAIL)])

        pltpu.sync_copy(gidx_h.at[pl.ds(base, per_t)], gi_v)
        pltpu.sync_copy(sidx_h.at[pl.ds(base, per_t)], si_v)
        if with_coef:
            pltpu.sync_copy(coef_h.at[pl.ds(base, per_t)], cf_v)
        plsc.subcore_barrier()

        def gather(c, rows, sem):
            off = pl.multiple_of(c * ch, 8)
            pltpu.async_copy(table_h.at[gi_v.at[pl.ds(off, ch)]], rows, sem)

        def gather_wait(c, rows, sem):
            off = pl.multiple_of(c * ch, 8)
            pltpu.make_async_copy(table_h.at[gi_v.at[pl.ds(off, ch)]],
                                  rows, sem).wait()

        def scatter(c, rows, sem):
            off = pl.multiple_of(c * ch, 8)
            pltpu.async_copy(rows, acc_s.at[si_v.at[pl.ds(off, ch)]],
                             sem, add=True)

        def scatter_wait(c, rows, sem):
            off = pl.multiple_of(c * ch, 8)
            pltpu.make_async_copy(rows, acc_s.at[si_v.at[pl.ds(off, ch)]],
                                  sem).wait()

        def scale(c, rows):
            if not with_coef:
                return
            cb0 = c * ch

            def row(i, _):
                i0 = 2 * i
                ca = plsc.load_gather(cf_v, [jnp.full((16,), cb0 + i0, i32)])
                cb = plsc.load_gather(cf_v, [jnp.full((16,), cb0 + i0 + 1, i32)])
                for m in range(d // 16):
                    sla = (i0, pl.ds(16 * m, 16))
                    slb = (i0 + 1, pl.ds(16 * m, 16))
                    rows[sla] = rows[sla] * ca
                    rows[slb] = rows[slb] * cb
                return 0

            lax.fori_loop(0, ch // 2, row, 0)

        npair = nch // 2
        gather(0, rows0, g0)

        def pair(kk, _):
            c0 = 2 * kk

            @pl.when(kk > 0)
            def _():
                scatter_wait(c0 - 1, rows1, s1)

            gather(c0 + 1, rows1, g1)
            gather_wait(c0, rows0, g0)
            scale(c0, rows0)
            scatter(c0, rows0, s0)
            gather_wait(c0 + 1, rows1, g1)
            scale(c0 + 1, rows1)
            scatter_wait(c0, rows0, s0)

            @pl.when(kk + 1 < npair)
            def _():
                gather(c0 + 2, rows0, g0)

            scatter(c0 + 1, rows1, s1)
            return 0

        lax.fori_loop(0, npair, pair, 0)
        scatter_wait(2 * npair - 1, rows1, s1)
        if nch % 2 == 1:
            gather(nch - 1, rows0, g0)
            gather_wait(nch - 1, rows0, g0)
            scale(nch - 1, rows0)
            scatter(nch - 1, rows0, s0)
            scatter_wait(nch - 1, rows0, s0)
        plsc.subcore_barrier()

        @pl.when(sid < 15)
        def _():
            pltpu.sync_copy(acc_s.at[pl.ds(r0, STRIPE)],
                            out_h.at[cid, pl.ds(r0, STRIPE)])

        @pl.when(sid == 15)
        def _():
            pltpu.sync_copy(acc_s.at[pl.ds(15 * STRIPE, TAIL)],
                            out_h.at[cid, pl.ds(15 * STRIPE, TAIL)])

    if with_coef:
        def body_c(table_h, gidx_h, sidx_h, coef_h, z2_h, out_h,
                   gi_v, si_v, rows0, rows1, cf_v, acc_s, g0, g1, s0, s1):
            body(table_h, gidx_h, sidx_h, coef_h, z2_h, out_h,
                 gi_v, si_v, rows0, rows1, cf_v, acc_s, g0, g1, s0, s1)

        k = functools.partial(
            pl.kernel,
            out_type=jax.ShapeDtypeStruct((NC, N, d), f32),
            mesh=_sc_mesh(),
            compiler_params=pltpu.CompilerParams(needs_layout_passes=False),
            scratch_types=scratch,
        )(body_c)
        return k(table, gidx, sidx, coef, zeros2d)

    def body_n(table_h, gidx_h, sidx_h, z2_h, out_h,
               gi_v, si_v, rows0, rows1, cf_v, acc_s, g0, g1, s0, s1):
        body(table_h, gidx_h, sidx_h, None, z2_h, out_h,
             gi_v, si_v, rows0, rows1, cf_v, acc_s, g0, g1, s0, s1)

    k = functools.partial(
        pl.kernel,
        out_type=jax.ShapeDtypeStruct((NC, N, d), f32),
        mesh=_sc_mesh(),
        compiler_params=pltpu.CompilerParams(needs_layout_passes=False),
        scratch_types=scratch,
    )(body_n)
    return k(table, gidx, sidx, zeros2d)


def _gat_alpha(asrc, adst, src, dst, shift16, zvec):
    per_t = E // NW
    nv = per_t // 16

    @functools.partial(
        pl.kernel,
        out_type=(
            jax.ShapeDtypeStruct((E,), f32),
            jax.ShapeDtypeStruct((NC, N), f32),
        ),
        mesh=_sc_mesh(),
        compiler_params=pltpu.CompilerParams(needs_layout_passes=False),
        scratch_types=[
            pltpu.VMEM((N,), f32),
            pltpu.VMEM((N,), f32),
            pltpu.VMEM((per_t,), i32),
            pltpu.VMEM((per_t,), i32),
            pltpu.VMEM((per_t,), f32),
            pltpu.VMEM((16,), f32),
            pltpu.VMEM_SHARED((N,), f32),
        ],
    )
    def k(as_h, ad_h, src_h, dst_h, sh_h, zv_h, ex_h, den_h,
          as_v, ad_v, src_v, dst_v, ex_v, sh_v, den_s):
        cid = lax.axis_index("c")
        sid = lax.axis_index("s")
        wid = cid * NS + sid
        base = wid * per_t

        @pl.when(sid == 0)
        def _():
            pltpu.sync_copy(zv_h, den_s)

        pltpu.sync_copy(as_h, as_v)
        pltpu.sync_copy(ad_h, ad_v)
        pltpu.sync_copy(src_h.at[pl.ds(base, per_t)], src_v)
        pltpu.sync_copy(dst_h.at[pl.ds(base, per_t)], dst_v)
        pltpu.sync_copy(sh_h, sh_v)
        plsc.subcore_barrier()
        cvec = sh_v[...]

        def body(j, _):
            sl = pl.ds(16 * j, 16)
            a = plsc.load_gather(as_v, [src_v[sl]])
            b = plsc.load_gather(ad_v, [dst_v[sl]])
            z = a + b
            z = jnp.where(z >= 0.0, z, 0.2 * z)
            ex_v[sl] = jnp.exp(z - cvec)
            return 0

        lax.fori_loop(0, nv, body, 0)
        pltpu.sync_copy(ex_v, ex_h.at[pl.ds(base, per_t)])
        pltpu.sync_copy(ex_v, den_s.at[dst_v], add=True)
        plsc.subcore_barrier()

        @pl.when(sid == 0)
        def _():
            pltpu.sync_copy(den_s, den_h.at[cid])

    return k(asrc, adst, src, dst, shift16, zvec)


_R = 1000


def _mm1(x, w1):
    def body(x_ref, w_ref, o_ref):
        o_ref[...] = jnp.dot(x_ref[...], w_ref[...], preferred_element_type=f32)

    return pl.pallas_call(
        body,
        grid=(N // _R,),
        in_specs=[
            pl.BlockSpec((_R, D_IN), lambda i: (i, 0)),
            pl.BlockSpec((D_IN, D_H), lambda i: (0, 0)),
        ],
        out_specs=pl.BlockSpec((_R, D_H), lambda i: (i, 0)),
        out_shape=jax.ShapeDtypeStruct((N, D_H), f32),
    )(x, w1)


def _scale_e(e_p, de_p):
    def body(e_ref, d_ref, o_ref):
        de = d_ref[0] + d_ref[1]
        inv = jnp.where(de > 0.0, 1.0 / de, 0.0)
        o_ref[...] = (e_ref[0] + e_ref[1]) * inv

    return pl.pallas_call(
        body,
        grid=(N // _R,),
        in_specs=[
            pl.BlockSpec((NC, _R, D_H), lambda i: (0, i, 0)),
            pl.BlockSpec((NC, _R, 1), lambda i: (0, i, 0)),
        ],
        out_specs=pl.BlockSpec((_R, D_H), lambda i: (i, 0)),
        out_shape=jax.ShapeDtypeStruct((N, D_H), f32),
    )(e_p, de_p)


def _h1_mm2(o_p, dn_p, b1, w2, a_s, a_d):
    def body(o_ref, d_ref, b_ref, w_ref, s_ref, t_ref, xl_ref, as_ref, ad_ref):
        dn = d_ref[0] + d_ref[1]
        inv = jnp.where(dn > 0.0, 1.0 / dn, 0.0)
        h = jnp.maximum((o_ref[0] + o_ref[1]) * inv + b_ref[...], 0.0)
        xl = jnp.dot(h, w_ref[...], preferred_element_type=f32)
        xl_ref[...] = xl
        as_ref[...] = jnp.sum(xl * s_ref[...], axis=-1, keepdims=True)
        ad_ref[...] = jnp.sum(xl * t_ref[...], axis=-1, keepdims=True)

    return pl.pallas_call(
        body,
        grid=(N // _R,),
        in_specs=[
            pl.BlockSpec((NC, _R, D_H), lambda i: (0, i, 0)),
            pl.BlockSpec((NC, _R, 1), lambda i: (0, i, 0)),
            pl.BlockSpec((1, D_H), lambda i: (0, 0)),
            pl.BlockSpec((D_H, D_H), lambda i: (0, 0)),
            pl.BlockSpec((1, D_H), lambda i: (0, 0)),
            pl.BlockSpec((1, D_H), lambda i: (0, 0)),
        ],
        out_specs=[
            pl.BlockSpec((_R, D_H), lambda i: (i, 0)),
            pl.BlockSpec((_R, 1), lambda i: (i, 0)),
            pl.BlockSpec((_R, 1), lambda i: (i, 0)),
        ],
        out_shape=[
            jax.ShapeDtypeStruct((N, D_H), f32),
            jax.ShapeDtypeStruct((N, 1), f32),
            jax.ShapeDtypeStruct((N, 1), f32),
        ],
    )(o_p, dn_p, b1, w2, a_s, a_d)


def _self_terms(a_s, a_d):
    def body(s_ref, t_ref, sh_ref, ex_ref):
        m = jnp.max(s_ref[...]) + jnp.max(t_ref[...])
        c = jnp.where(m >= 0.0, m, 0.2 * m)
        sh_ref[...] = jnp.full((1, 16), c, f32)
        z = s_ref[...] + t_ref[...]
        z = jnp.where(z >= 0.0, z, 0.2 * z)
        ex_ref[...] = jnp.exp(z - c)

    return pl.pallas_call(
        body,
        in_specs=[
            pl.BlockSpec((N, 1), lambda: (0, 0)),
            pl.BlockSpec((N, 1), lambda: (0, 0)),
        ],
        out_specs=[
            pl.BlockSpec((1, 16), lambda: (0, 0)),
            pl.BlockSpec((N, 1), lambda: (0, 0)),
        ],
        out_shape=[
            jax.ShapeDtypeStruct((1, 16), f32),
            jax.ShapeDtypeStruct((N, 1), f32),
        ],
    )(a_s, a_d)


def _gat_comb(p, exs, xl, den_p, b, w_next=None, a_s=None, a_d=None, d_out=None,
              out_slice=None):
    d = xl.shape[-1]
    fused = w_next is not None
    o = out_slice if out_slice is not None else d

    def body(p_ref, e_ref, x_ref, d_ref, b_ref, *rest):
        if fused:
            w_ref, s_ref, t_ref, xl_ref, as_ref, ad_ref = rest
        else:
            (h_ref,) = rest
        den = d_ref[0] + d_ref[1] + e_ref[...]
        acc = p_ref[0] + p_ref[1] + e_ref[...] * x_ref[...]
        h = jnp.maximum(acc / (den + 1e-16) + b_ref[...], 0.0)
        if fused:
            xl = jnp.dot(h, w_ref[...], preferred_element_type=f32)
            xl_ref[...] = xl
            as_ref[...] = jnp.sum(xl * s_ref[...], axis=-1, keepdims=True)
            ad_ref[...] = jnp.sum(xl * t_ref[...], axis=-1, keepdims=True)
        else:
            h_ref[...] = h[:, :o]

    in_specs = [
        pl.BlockSpec((NC, _R, d), lambda i: (0, i, 0)),
        pl.BlockSpec((_R, 1), lambda i: (i, 0)),
        pl.BlockSpec((_R, d), lambda i: (i, 0)),
        pl.BlockSpec((NC, _R, 1), lambda i: (0, i, 0)),
        pl.BlockSpec((1, d), lambda i: (0, 0)),
    ]
    args = [p, exs, xl, den_p, b]
    if fused:
        in_specs += [
            pl.BlockSpec((d, d_out), lambda i: (0, 0)),
            pl.BlockSpec((1, d_out), lambda i: (0, 0)),
            pl.BlockSpec((1, d_out), lambda i: (0, 0)),
        ]
        args += [w_next, a_s, a_d]
        out_specs = [
            pl.BlockSpec((_R, d_out), lambda i: (i, 0)),
            pl.BlockSpec((_R, 1), lambda i: (i, 0)),
            pl.BlockSpec((_R, 1), lambda i: (i, 0)),
        ]
        out_shape = [
            jax.ShapeDtypeStruct((N, d_out), f32),
            jax.ShapeDtypeStruct((N, 1), f32),
            jax.ShapeDtypeStruct((N, 1), f32),
        ]
    else:
        out_specs = pl.BlockSpec((_R, o), lambda i: (i, 0))
        out_shape = jax.ShapeDtypeStruct((N, o), f32)

    return pl.pallas_call(
        body,
        grid=(N // _R,),
        in_specs=in_specs,
        out_specs=out_specs,
        out_shape=out_shape,
    )(*args)


def kernel(x, edge_index, hyperedge_index, W1, b1, W2, att_src2, att_dst2, b2,
           W3, att_src3, att_dst3, b3):
    nidx = hyperedge_index[0]
    eidx = hyperedge_index[1]
    src = edge_index[0]
    dst = edge_index[1]

    zvec = jnp.zeros((N,), f32)
    z2h = jnp.zeros((N, D_H), f32)
    z2o = jnp.zeros((N, D_OUT), f32)
    ones_t = jnp.ones((NNZ // NW,), f32)

    x1 = _mm1(x, W1)
    dn_p, de_p = _degrees(nidx, eidx, ones_t, zvec)
    e_p = _hop(x1, nidx, eidx, None, z2h, D_H, NNZ, 40)
    e_s = _scale_e(e_p, de_p.reshape(NC, N, 1))
    o_p = _hop(e_s, eidx, nidx, None, z2h, D_H, NNZ, 40)

    xl2, as2, ad2 = _h1_mm2(
        o_p, dn_p.reshape(NC, N, 1), b1.reshape(1, D_H), W2,
        att_src2.reshape(1, D_H), att_dst2.reshape(1, D_H))
    sh2, exs2 = _self_terms(as2, ad2)
    ex2, den2p = _gat_alpha(
        as2.reshape(N), ad2.reshape(N), src, dst, sh2.reshape(16), zvec)
    p2 = _hop(xl2, src, dst, ex2, z2h, D_H, E, 80)

    pad = D_H - D_OUT
    w3p = jnp.pad(W3, ((0, 0), (0, pad)))
    xl3, as3, ad3 = _gat_comb(
        p2, exs2, xl2, den2p.reshape(NC, N, 1), b2.reshape(1, D_H),
        w_next=w3p, a_s=jnp.pad(att_src3, (0, pad)).reshape(1, D_H),
        a_d=jnp.pad(att_dst3, (0, pad)).reshape(1, D_H), d_out=D_H)
    sh3, exs3 = _self_terms(as3, ad3)
    ex3, den3p = _gat_alpha(
        as3.reshape(N), ad3.reshape(N), src, dst, sh3.reshape(16), zvec)
    p3 = _hop(xl3, src, dst, ex3, z2h, D_H, E, 80)

    return _gat_comb(p3, exs3, xl3, den3p.reshape(NC, N, 1),
                     jnp.pad(b3, (0, pad)).reshape(1, D_H), out_slice=D_OUT)


# speedup vs baseline: 39.1850x; 1.1132x over previous
import functools

import jax
import jax.numpy as jnp
from jax import lax
from jax.experimental import pallas as pl
from jax.experimental.pallas import tpu as pltpu
from jax.experimental.pallas import tpu_sc as plsc

N = 10000
E = 320000
NNZ = 160000
D_IN = 128
D_H = 128
D_OUT = 64

NC = 2
NS = 16
NW = NC * NS
STRIPE = 632
TAIL = N - 15 * STRIPE
CHUNK = 200

f32 = jnp.float32
i32 = jnp.int32


def _sc_mesh():
    return plsc.VectorSubcoreMesh(
        core_axis_name="c", subcore_axis_name="s", num_cores=NC, num_subcores=NS
    )


def _degrees(nidx, eidx, ones_t, zvec):
    per_t = NNZ // NW

    @functools.partial(
        pl.kernel,
        out_type=(
            jax.ShapeDtypeStruct((NC, N), f32),
            jax.ShapeDtypeStruct((NC, N), f32),
        ),
        mesh=_sc_mesh(),
        scratch_types=[
            pltpu.VMEM((per_t,), i32),
            pltpu.VMEM((per_t,), f32),
            pltpu.VMEM_SHARED((N,), f32),
            pltpu.VMEM_SHARED((N,), f32),
        ],
    )
    def k(nidx_h, eidx_h, ones_h, zvec_h, dn_h, de_h, idx_v, ones_v, dn_s, de_s):
        cid = lax.axis_index("c")
        sid = lax.axis_index("s")
        wid = cid * NS + sid
        base = wid * per_t

        @pl.when(sid == 0)
        def _():
            pltpu.sync_copy(zvec_h, dn_s)
            pltpu.sync_copy(zvec_h, de_s)

        pltpu.sync_copy(ones_h, ones_v)
        pltpu.sync_copy(nidx_h.at[pl.ds(base, per_t)], idx_v)
        plsc.subcore_barrier()
        pltpu.sync_copy(ones_v, dn_s.at[idx_v], add=True)
        pltpu.sync_copy(eidx_h.at[pl.ds(base, per_t)], idx_v)
        pltpu.sync_copy(ones_v, de_s.at[idx_v], add=True)
        plsc.subcore_barrier()

        @pl.when(sid == 0)
        def _():
            pltpu.sync_copy(dn_s, dn_h.at[cid])
            pltpu.sync_copy(de_s, de_h.at[cid])

    return k(nidx, eidx, ones_t, zvec)


def _hop(table, gidx, sidx, coef, zeros2d, d, n_edges, ch):
    per_t = n_edges // NW
    nch = per_t // ch
    tail = per_t - nch * ch
    assert ch % 8 == 0 and tail % 8 == 0
    with_coef = coef is not None

    scratch = [
        pltpu.VMEM((per_t,), i32),
        pltpu.VMEM((per_t,), i32),
        pltpu.VMEM((ch, d), f32),
        pltpu.VMEM((ch, d), f32),
    ] + ([pltpu.VMEM((per_t,), f32)] if with_coef else []) + [
        pltpu.VMEM_SHARED((N, d), f32),
        pltpu.SemaphoreType.DMA,
        pltpu.SemaphoreType.DMA,
        pltpu.SemaphoreType.DMA,
        pltpu.SemaphoreType.DMA,
    ]

    def body(table_h, gidx_h, sidx_h, coef_h, z2_h, out_h,
             gi_v, si_v, rows0, rows1, cf_v, acc_s, g0, g1, s0, s1):
        cid = lax.axis_index("c")
        sid = lax.axis_index("s")
        wid = cid * NS + sid
        base = wid * per_t
        r0 = pl.multiple_of(sid * STRIPE, 8)

        @pl.when(sid < 15)
        def _():
            pltpu.sync_copy(z2_h.at[pl.ds(r0, STRIPE)], acc_s.at[pl.ds(r0, STRIPE)])

        @pl.when(sid == 15)
        def _():
            pltpu.sync_copy(z2_h.at[pl.ds(15 * STRIPE, TAIL)],
                            acc_s.at[pl.ds(15 * STRIPE, TAIL)])

        pltpu.sync_copy(gidx_h.at[pl.ds(base, per_t)], gi_v)
        pltpu.sync_copy(sidx_h.at[pl.ds(base, per_t)], si_v)
        if with_coef:
            pltpu.sync_copy(coef_h.at[pl.ds(base, per_t)], cf_v)
        plsc.subcore_barrier()

        def gather(c, rows, sem):
            off = pl.multiple_of(c * ch, 8)
            pltpu.async_copy(table_h.at[gi_v.at[pl.ds(off, ch)]], rows, sem)

        def gather_wait(c, rows, sem):
            off = pl.multiple_of(c * ch, 8)
            pltpu.make_async_copy(table_h.at[gi_v.at[pl.ds(off, ch)]],
                                  rows, sem).wait()

        def scatter(c, rows, sem):
            off = pl.multiple_of(c * ch, 8)
            pltpu.async_copy(rows, acc_s.at[si_v.at[pl.ds(off, ch)]],
                             sem, add=True)

        def scatter_wait(c, rows, sem):
            off = pl.multiple_of(c * ch, 8)
            pltpu.make_async_copy(rows, acc_s.at[si_v.at[pl.ds(off, ch)]],
                                  sem).wait()

        def scale(c, rows, n):
            if not with_coef:
                return
            cb0 = c * ch

            @plsc.parallel_loop(0, n, 1, unroll=4)
            def _(i):
                cb = plsc.load_gather(cf_v, [jnp.full((16,), cb0 + i, i32)])
                for m in range(d // 16):
                    sl = (i, pl.ds(16 * m, 16))
                    rows[sl] = rows[sl] * cb

        npair = nch // 2
        gather(0, rows0, g0)

        def pair(kk, _):
            c0 = 2 * kk

            @pl.when(kk > 0)
            def _():
                scatter_wait(c0 - 1, rows1, s1)

            gather(c0 + 1, rows1, g1)
            gather_wait(c0, rows0, g0)
            scale(c0, rows0, ch)
            scatter(c0, rows0, s0)
            gather_wait(c0 + 1, rows1, g1)
            scale(c0 + 1, rows1, ch)
            scatter_wait(c0, rows0, s0)

            @pl.when(kk + 1 < npair)
            def _():
                gather(c0 + 2, rows0, g0)

            scatter(c0 + 1, rows1, s1)
            return 0

        lax.fori_loop(0, npair, pair, 0)
        scatter_wait(2 * npair - 1, rows1, s1)
        if nch % 2 == 1:
            gather(nch - 1, rows0, g0)
            gather_wait(nch - 1, rows0, g0)
            scale(nch - 1, rows0, ch)
            scatter(nch - 1, rows0, s0)
            scatter_wait(nch - 1, rows0, s0)
        if tail > 0:
            toff = pl.multiple_of(nch * ch, 8)
            tr = rows1.at[pl.ds(0, tail)]
            pltpu.async_copy(table_h.at[gi_v.at[pl.ds(toff, tail)]], tr, g1)
            pltpu.make_async_copy(table_h.at[gi_v.at[pl.ds(toff, tail)]],
                                  tr, g1).wait()
            scale(nch, rows1, tail)
            pltpu.async_copy(tr, acc_s.at[si_v.at[pl.ds(toff, tail)]],
                             s1, add=True)
            pltpu.make_async_copy(tr, acc_s.at[si_v.at[pl.ds(toff, tail)]],
                                  s1).wait()
        plsc.subcore_barrier()

        @pl.when(sid < 15)
        def _():
            pltpu.sync_copy(acc_s.at[pl.ds(r0, STRIPE)],
                            out_h.at[cid, pl.ds(r0, STRIPE)])

        @pl.when(sid == 15)
        def _():
            pltpu.sync_copy(acc_s.at[pl.ds(15 * STRIPE, TAIL)],
                            out_h.at[cid, pl.ds(15 * STRIPE, TAIL)])

    if with_coef:
        def body_c(table_h, gidx_h, sidx_h, coef_h, z2_h, out_h,
                   gi_v, si_v, rows0, rows1, cf_v, acc_s, g0, g1, s0, s1):
            body(table_h, gidx_h, sidx_h, coef_h, z2_h, out_h,
                 gi_v, si_v, rows0, rows1, cf_v, acc_s, g0, g1, s0, s1)

        k = functools.partial(
            pl.kernel,
            out_type=jax.ShapeDtypeStruct((NC, N, d), f32),
            mesh=_sc_mesh(),
            compiler_params=pltpu.CompilerParams(needs_layout_passes=False),
            scratch_types=scratch,
        )(body_c)
        return k(table, gidx, sidx, coef, zeros2d)

    def body_n(table_h, gidx_h, sidx_h, z2_h, out_h,
               gi_v, si_v, rows0, rows1, acc_s, g0, g1, s0, s1):
        body(table_h, gidx_h, sidx_h, None, z2_h, out_h,
             gi_v, si_v, rows0, rows1, None, acc_s, g0, g1, s0, s1)

    k = functools.partial(
        pl.kernel,
        out_type=jax.ShapeDtypeStruct((NC, N, d), f32),
        mesh=_sc_mesh(),
        compiler_params=pltpu.CompilerParams(needs_layout_passes=False),
        scratch_types=scratch,
    )(body_n)
    return k(table, gidx, sidx, zeros2d)


def _gat_alpha(asrc, adst, src, dst, shift16, zvec):
    per_t = E // NW
    nv = per_t // 16

    @functools.partial(
        pl.kernel,
        out_type=(
            jax.ShapeDtypeStruct((E,), f32),
            jax.ShapeDtypeStruct((NC, N), f32),
        ),
        mesh=_sc_mesh(),
        compiler_params=pltpu.CompilerParams(needs_layout_passes=False),
        scratch_types=[
            pltpu.VMEM((N,), f32),
            pltpu.VMEM((N,), f32),
            pltpu.VMEM((per_t,), i32),
            pltpu.VMEM((per_t,), i32),
            pltpu.VMEM((per_t,), f32),
            pltpu.VMEM((16,), f32),
            pltpu.VMEM_SHARED((N,), f32),
        ],
    )
    def k(as_h, ad_h, src_h, dst_h, sh_h, zv_h, ex_h, den_h,
          as_v, ad_v, src_v, dst_v, ex_v, sh_v, den_s):
        cid = lax.axis_index("c")
        sid = lax.axis_index("s")
        wid = cid * NS + sid
        base = wid * per_t

        @pl.when(sid == 0)
        def _():
            pltpu.sync_copy(zv_h, den_s)

        pltpu.sync_copy(as_h, as_v)
        pltpu.sync_copy(ad_h, ad_v)
        pltpu.sync_copy(src_h.at[pl.ds(base, per_t)], src_v)
        pltpu.sync_copy(dst_h.at[pl.ds(base, per_t)], dst_v)
        pltpu.sync_copy(sh_h, sh_v)
        plsc.subcore_barrier()
        cvec = sh_v[...]

        def body(j, _):
            sl = pl.ds(16 * j, 16)
            a = plsc.load_gather(as_v, [src_v[sl]])
            b = plsc.load_gather(ad_v, [dst_v[sl]])
            z = a + b
            z = jnp.where(z >= 0.0, z, 0.2 * z)
            ex_v[sl] = jnp.exp(z - cvec)
            return 0

        lax.fori_loop(0, nv, body, 0)
        pltpu.sync_copy(ex_v, ex_h.at[pl.ds(base, per_t)])
        pltpu.sync_copy(ex_v, den_s.at[dst_v], add=True)
        plsc.subcore_barrier()

        @pl.when(sid == 0)
        def _():
            pltpu.sync_copy(den_s, den_h.at[cid])

    return k(asrc, adst, src, dst, shift16, zvec)


_R = 1000


def _mm1(x, w1):
    def body(x_ref, w_ref, o_ref):
        o_ref[...] = jnp.dot(x_ref[...], w_ref[...], preferred_element_type=f32)

    return pl.pallas_call(
        body,
        grid=(N // _R,),
        in_specs=[
            pl.BlockSpec((_R, D_IN), lambda i: (i, 0)),
            pl.BlockSpec((D_IN, D_H), lambda i: (0, 0)),
        ],
        out_specs=pl.BlockSpec((_R, D_H), lambda i: (i, 0)),
        out_shape=jax.ShapeDtypeStruct((N, D_H), f32),
    )(x, w1)


def _scale_e(e_p, de_p):
    def body(e_ref, d_ref, o_ref):
        de = d_ref[0] + d_ref[1]
        inv = jnp.where(de > 0.0, 1.0 / de, 0.0)
        o_ref[...] = (e_ref[0] + e_ref[1]) * inv

    return pl.pallas_call(
        body,
        grid=(N // _R,),
        in_specs=[
            pl.BlockSpec((NC, _R, D_H), lambda i: (0, i, 0)),
            pl.BlockSpec((NC, _R, 1), lambda i: (0, i, 0)),
        ],
        out_specs=pl.BlockSpec((_R, D_H), lambda i: (i, 0)),
        out_shape=jax.ShapeDtypeStruct((N, D_H), f32),
    )(e_p, de_p)


def _h1_mm2(o_p, dn_p, b1, w2, a_s, a_d):
    def body(o_ref, d_ref, b_ref, w_ref, s_ref, t_ref, xl_ref, as_ref, ad_ref):
        dn = d_ref[0] + d_ref[1]
        inv = jnp.where(dn > 0.0, 1.0 / dn, 0.0)
        h = jnp.maximum((o_ref[0] + o_ref[1]) * inv + b_ref[...], 0.0)
        xl = jnp.dot(h, w_ref[...], preferred_element_type=f32)
        xl_ref[...] = xl
        as_ref[...] = jnp.sum(xl * s_ref[...], axis=-1, keepdims=True)
        ad_ref[...] = jnp.sum(xl * t_ref[...], axis=-1, keepdims=True)

    return pl.pallas_call(
        body,
        grid=(N // _R,),
        in_specs=[
            pl.BlockSpec((NC, _R, D_H), lambda i: (0, i, 0)),
            pl.BlockSpec((NC, _R, 1), lambda i: (0, i, 0)),
            pl.BlockSpec((1, D_H), lambda i: (0, 0)),
            pl.BlockSpec((D_H, D_H), lambda i: (0, 0)),
            pl.BlockSpec((1, D_H), lambda i: (0, 0)),
            pl.BlockSpec((1, D_H), lambda i: (0, 0)),
        ],
        out_specs=[
            pl.BlockSpec((_R, D_H), lambda i: (i, 0)),
            pl.BlockSpec((_R, 1), lambda i: (i, 0)),
            pl.BlockSpec((_R, 1), lambda i: (i, 0)),
        ],
        out_shape=[
            jax.ShapeDtypeStruct((N, D_H), f32),
            jax.ShapeDtypeStruct((N, 1), f32),
            jax.ShapeDtypeStruct((N, 1), f32),
        ],
    )(o_p, dn_p, b1, w2, a_s, a_d)


def _self_terms(a_s, a_d):
    def body(s_ref, t_ref, sh_ref, ex_ref):
        m = jnp.max(s_ref[...]) + jnp.max(t_ref[...])
        c = jnp.where(m >= 0.0, m, 0.2 * m)
        sh_ref[...] = jnp.full((1, 16), c, f32)
        z = s_ref[...] + t_ref[...]
        z = jnp.where(z >= 0.0, z, 0.2 * z)
        ex_ref[...] = jnp.exp(z - c)

    return pl.pallas_call(
        body,
        in_specs=[
            pl.BlockSpec((N, 1), lambda: (0, 0)),
            pl.BlockSpec((N, 1), lambda: (0, 0)),
        ],
        out_specs=[
            pl.BlockSpec((1, 16), lambda: (0, 0)),
            pl.BlockSpec((N, 1), lambda: (0, 0)),
        ],
        out_shape=[
            jax.ShapeDtypeStruct((1, 16), f32),
            jax.ShapeDtypeStruct((N, 1), f32),
        ],
    )(a_s, a_d)


def _gat_comb(p, exs, xl, den_p, b, w_next=None, a_s=None, a_d=None, d_out=None,
              out_slice=None):
    d = xl.shape[-1]
    fused = w_next is not None
    o = out_slice if out_slice is not None else d

    def body(p_ref, e_ref, x_ref, d_ref, b_ref, *rest):
        if fused:
            w_ref, s_ref, t_ref, xl_ref, as_ref, ad_ref = rest
        else:
            (h_ref,) = rest
        den = d_ref[0] + d_ref[1] + e_ref[...]
        acc = p_ref[0] + p_ref[1] + e_ref[...] * x_ref[...]
        h = jnp.maximum(acc / (den + 1e-16) + b_ref[...], 0.0)
        if fused:
            xl = jnp.dot(h, w_ref[...], preferred_element_type=f32)
            xl_ref[...] = xl
            as_ref[...] = jnp.sum(xl * s_ref[...], axis=-1, keepdims=True)
            ad_ref[...] = jnp.sum(xl * t_ref[...], axis=-1, keepdims=True)
        else:
            h_ref[...] = h[:, :o]

    in_specs = [
        pl.BlockSpec((NC, _R, d), lambda i: (0, i, 0)),
        pl.BlockSpec((_R, 1), lambda i: (i, 0)),
        pl.BlockSpec((_R, d), lambda i: (i, 0)),
        pl.BlockSpec((NC, _R, 1), lambda i: (0, i, 0)),
        pl.BlockSpec((1, d), lambda i: (0, 0)),
    ]
    args = [p, exs, xl, den_p, b]
    if fused:
        in_specs += [
            pl.BlockSpec((d, d_out), lambda i: (0, 0)),
            pl.BlockSpec((1, d_out), lambda i: (0, 0)),
            pl.BlockSpec((1, d_out), lambda i: (0, 0)),
        ]
        args += [w_next, a_s, a_d]
        out_specs = [
            pl.BlockSpec((_R, d_out), lambda i: (i, 0)),
            pl.BlockSpec((_R, 1), lambda i: (i, 0)),
            pl.BlockSpec((_R, 1), lambda i: (i, 0)),
        ]
        out_shape = [
            jax.ShapeDtypeStruct((N, d_out), f32),
            jax.ShapeDtypeStruct((N, 1), f32),
            jax.ShapeDtypeStruct((N, 1), f32),
        ]
    else:
        out_specs = pl.BlockSpec((_R, o), lambda i: (i, 0))
        out_shape = jax.ShapeDtypeStruct((N, o), f32)

    return pl.pallas_call(
        body,
        grid=(N // _R,),
        in_specs=in_specs,
        out_specs=out_specs,
        out_shape=out_shape,
    )(*args)


def kernel(x, edge_index, hyperedge_index, W1, b1, W2, att_src2, att_dst2, b2,
           W3, att_src3, att_dst3, b3):
    nidx = hyperedge_index[0]
    eidx = hyperedge_index[1]
    src = edge_index[0]
    dst = edge_index[1]

    zvec = jnp.zeros((N,), f32)
    z2h = jnp.zeros((N, D_H), f32)
    z2o = jnp.zeros((N, D_OUT), f32)
    ones_t = jnp.ones((NNZ // NW,), f32)

    x1 = _mm1(x, W1)
    dn_p, de_p = _degrees(nidx, eidx, ones_t, zvec)
    e_p = _hop(x1, nidx, eidx, None, z2h, D_H, NNZ, 128)
    e_s = _scale_e(e_p, de_p.reshape(NC, N, 1))
    o_p = _hop(e_s, eidx, nidx, None, z2h, D_H, NNZ, 128)

    xl2, as2, ad2 = _h1_mm2(
        o_p, dn_p.reshape(NC, N, 1), b1.reshape(1, D_H), W2,
        att_src2.reshape(1, D_H), att_dst2.reshape(1, D_H))
    sh2, exs2 = _self_terms(as2, ad2)
    ex2, den2p = _gat_alpha(
        as2.reshape(N), ad2.reshape(N), src, dst, sh2.reshape(16), zvec)
    p2 = _hop(xl2, src, dst, ex2, z2h, D_H, E, 80)

    pad = D_H - D_OUT
    w3p = jnp.pad(W3, ((0, 0), (0, pad)))
    xl3, as3, ad3 = _gat_comb(
        p2, exs2, xl2, den2p.reshape(NC, N, 1), b2.reshape(1, D_H),
        w_next=w3p, a_s=jnp.pad(att_src3, (0, pad)).reshape(1, D_H),
        a_d=jnp.pad(att_dst3, (0, pad)).reshape(1, D_H), d_out=D_H)
    sh3, exs3 = _self_terms(as3, ad3)
    ex3, den3p = _gat_alpha(
        as3.reshape(N), ad3.reshape(N), src, dst, sh3.reshape(16), zvec)
    p3 = _hop(xl3, src, dst, ex3, z2h, D_H, E, 80)

    return _gat_comb(p3, exs3, xl3, den3p.reshape(NC, N, 1),
                     jnp.pad(b3, (0, pad)).reshape(1, D_H), out_slice=D_OUT)


# speedup vs baseline: 39.7935x; 1.0155x over previous
import functools

import jax
import jax.numpy as jnp
from jax import lax
from jax.experimental import pallas as pl
from jax.experimental.pallas import tpu as pltpu
from jax.experimental.pallas import tpu_sc as plsc

N = 10000
E = 320000
NNZ = 160000
D_IN = 128
D_H = 128
D_OUT = 64

NC = 2
NS = 16
NW = NC * NS
STRIPE = 632
TAIL = N - 15 * STRIPE
CHUNK = 200

f32 = jnp.float32
i32 = jnp.int32


def _sc_mesh():
    return plsc.VectorSubcoreMesh(
        core_axis_name="c", subcore_axis_name="s", num_cores=NC, num_subcores=NS
    )


def _degrees(nidx, eidx, ones_t, zvec):
    per_t = NNZ // NW

    @functools.partial(
        pl.kernel,
        out_type=(
            jax.ShapeDtypeStruct((NC, N), f32),
            jax.ShapeDtypeStruct((NC, N), f32),
        ),
        mesh=_sc_mesh(),
        scratch_types=[
            pltpu.VMEM((per_t,), i32),
            pltpu.VMEM((per_t,), f32),
            pltpu.VMEM_SHARED((N,), f32),
            pltpu.VMEM_SHARED((N,), f32),
        ],
    )
    def k(nidx_h, eidx_h, ones_h, zvec_h, dn_h, de_h, idx_v, ones_v, dn_s, de_s):
        cid = lax.axis_index("c")
        sid = lax.axis_index("s")
        wid = cid * NS + sid
        base = wid * per_t

        @pl.when(sid == 0)
        def _():
            pltpu.sync_copy(zvec_h, dn_s)
            pltpu.sync_copy(zvec_h, de_s)

        pltpu.sync_copy(ones_h, ones_v)
        pltpu.sync_copy(nidx_h.at[pl.ds(base, per_t)], idx_v)
        plsc.subcore_barrier()
        pltpu.sync_copy(ones_v, dn_s.at[idx_v], add=True)
        pltpu.sync_copy(eidx_h.at[pl.ds(base, per_t)], idx_v)
        pltpu.sync_copy(ones_v, de_s.at[idx_v], add=True)
        plsc.subcore_barrier()

        @pl.when(sid == 0)
        def _():
            pltpu.sync_copy(dn_s, dn_h.at[cid])
            pltpu.sync_copy(de_s, de_h.at[cid])

    return k(nidx, eidx, ones_t, zvec)


def _hop(table, gidx, sidx, coef, zeros2d, d, n_edges, ch):
    per_t = n_edges // NW
    nch = per_t // ch
    tail = per_t - nch * ch
    assert ch % 8 == 0 and tail % 8 == 0
    with_coef = coef is not None

    scratch = [
        pltpu.VMEM((per_t,), i32),
        pltpu.VMEM((per_t,), i32),
        pltpu.VMEM((ch, d), f32),
        pltpu.VMEM((ch, d), f32),
    ] + ([pltpu.VMEM((per_t,), f32)] if with_coef else []) + [
        pltpu.VMEM_SHARED((N, d), f32),
        pltpu.SemaphoreType.DMA,
        pltpu.SemaphoreType.DMA,
        pltpu.SemaphoreType.DMA,
        pltpu.SemaphoreType.DMA,
    ]

    def body(table_h, gidx_h, sidx_h, coef_h, z2_h, out_h,
             gi_v, si_v, rows0, rows1, cf_v, acc_s, g0, g1, s0, s1):
        cid = lax.axis_index("c")
        sid = lax.axis_index("s")
        wid = cid * NS + sid
        base = wid * per_t
        r0 = pl.multiple_of(sid * STRIPE, 8)

        @pl.when(sid < 15)
        def _():
            pltpu.sync_copy(z2_h.at[pl.ds(r0, STRIPE)], acc_s.at[pl.ds(r0, STRIPE)])

        @pl.when(sid == 15)
        def _():
            pltpu.sync_copy(z2_h.at[pl.ds(15 * STRIPE, TAIL)],
                            acc_s.at[pl.ds(15 * STRIPE, TAIL)])

        pltpu.sync_copy(gidx_h.at[pl.ds(base, per_t)], gi_v)
        pltpu.sync_copy(sidx_h.at[pl.ds(base, per_t)], si_v)
        if with_coef:
            pltpu.sync_copy(coef_h.at[pl.ds(base, per_t)], cf_v)
        plsc.subcore_barrier()

        def gather(c, rows, sem):
            off = pl.multiple_of(c * ch, 8)
            pltpu.async_copy(table_h.at[gi_v.at[pl.ds(off, ch)]], rows, sem)

        def gather_wait(c, rows, sem):
            off = pl.multiple_of(c * ch, 8)
            pltpu.make_async_copy(table_h.at[gi_v.at[pl.ds(off, ch)]],
                                  rows, sem).wait()

        def scatter(c, rows, sem):
            off = pl.multiple_of(c * ch, 8)
            pltpu.async_copy(rows, acc_s.at[si_v.at[pl.ds(off, ch)]],
                             sem, add=True)

        def scatter_wait(c, rows, sem):
            off = pl.multiple_of(c * ch, 8)
            pltpu.make_async_copy(rows, acc_s.at[si_v.at[pl.ds(off, ch)]],
                                  sem).wait()

        def scale(c, rows, n):
            if not with_coef:
                return
            cb0 = c * ch

            @plsc.parallel_loop(0, n, 1, unroll=4)
            def _(i):
                cb = plsc.load_gather(cf_v, [jnp.full((16,), cb0 + i, i32)])
                for m in range(d // 16):
                    sl = (i, pl.ds(16 * m, 16))
                    rows[sl] = rows[sl] * cb

        npair = nch // 2
        gather(0, rows0, g0)

        def pair(kk, _):
            c0 = 2 * kk

            @pl.when(kk > 0)
            def _():
                scatter_wait(c0 - 1, rows1, s1)

            gather(c0 + 1, rows1, g1)
            gather_wait(c0, rows0, g0)
            scale(c0, rows0, ch)
            scatter(c0, rows0, s0)
            gather_wait(c0 + 1, rows1, g1)
            scale(c0 + 1, rows1, ch)
            scatter_wait(c0, rows0, s0)

            @pl.when(kk + 1 < npair)
            def _():
                gather(c0 + 2, rows0, g0)

            scatter(c0 + 1, rows1, s1)
            return 0

        lax.fori_loop(0, npair, pair, 0)
        scatter_wait(2 * npair - 1, rows1, s1)
        if nch % 2 == 1:
            gather(nch - 1, rows0, g0)
            gather_wait(nch - 1, rows0, g0)
            scale(nch - 1, rows0, ch)
            scatter(nch - 1, rows0, s0)
            scatter_wait(nch - 1, rows0, s0)
        if tail > 0:
            toff = pl.multiple_of(nch * ch, 8)
            tr = rows1.at[pl.ds(0, tail)]
            pltpu.async_copy(table_h.at[gi_v.at[pl.ds(toff, tail)]], tr, g1)
            pltpu.make_async_copy(table_h.at[gi_v.at[pl.ds(toff, tail)]],
                                  tr, g1).wait()
            scale(nch, rows1, tail)
            pltpu.async_copy(tr, acc_s.at[si_v.at[pl.ds(toff, tail)]],
                             s1, add=True)
            pltpu.make_async_copy(tr, acc_s.at[si_v.at[pl.ds(toff, tail)]],
                                  s1).wait()
        plsc.subcore_barrier()

        @pl.when(sid < 15)
        def _():
            pltpu.sync_copy(acc_s.at[pl.ds(r0, STRIPE)],
                            out_h.at[cid, pl.ds(r0, STRIPE)])

        @pl.when(sid == 15)
        def _():
            pltpu.sync_copy(acc_s.at[pl.ds(15 * STRIPE, TAIL)],
                            out_h.at[cid, pl.ds(15 * STRIPE, TAIL)])

    if with_coef:
        def body_c(table_h, gidx_h, sidx_h, coef_h, z2_h, out_h,
                   gi_v, si_v, rows0, rows1, cf_v, acc_s, g0, g1, s0, s1):
            body(table_h, gidx_h, sidx_h, coef_h, z2_h, out_h,
                 gi_v, si_v, rows0, rows1, cf_v, acc_s, g0, g1, s0, s1)

        k = functools.partial(
            pl.kernel,
            out_type=jax.ShapeDtypeStruct((NC, N, d), f32),
            mesh=_sc_mesh(),
            compiler_params=pltpu.CompilerParams(needs_layout_passes=False),
            scratch_types=scratch,
        )(body_c)
        return k(table, gidx, sidx, coef, zeros2d)

    def body_n(table_h, gidx_h, sidx_h, z2_h, out_h,
               gi_v, si_v, rows0, rows1, acc_s, g0, g1, s0, s1):
        body(table_h, gidx_h, sidx_h, None, z2_h, out_h,
             gi_v, si_v, rows0, rows1, None, acc_s, g0, g1, s0, s1)

    k = functools.partial(
        pl.kernel,
        out_type=jax.ShapeDtypeStruct((NC, N, d), f32),
        mesh=_sc_mesh(),
        compiler_params=pltpu.CompilerParams(needs_layout_passes=False),
        scratch_types=scratch,
    )(body_n)
    return k(table, gidx, sidx, zeros2d)


def _gat_alpha(asrc, adst, src, dst, zvec):
    per_t = E // NW
    nv = per_t // 16
    nvt = N // 16
    es_str = 640

    @functools.partial(
        pl.kernel,
        out_type=(
            jax.ShapeDtypeStruct((E,), f32),
            jax.ShapeDtypeStruct((NC, N), f32),
            jax.ShapeDtypeStruct((N,), f32),
        ),
        mesh=_sc_mesh(),
        compiler_params=pltpu.CompilerParams(needs_layout_passes=False),
        scratch_types=[
            pltpu.VMEM((N,), f32),
            pltpu.VMEM((N,), f32),
            pltpu.VMEM((per_t,), i32),
            pltpu.VMEM((per_t,), i32),
            pltpu.VMEM((per_t,), f32),
            pltpu.VMEM((es_str,), f32),
            pltpu.VMEM_SHARED((N,), f32),
        ],
    )
    def k(as_h, ad_h, src_h, dst_h, zv_h, ex_h, den_h, exs_h,
          as_v, ad_v, src_v, dst_v, ex_v, es_v, den_s):
        cid = lax.axis_index("c")
        sid = lax.axis_index("s")
        wid = cid * NS + sid
        base = wid * per_t

        @pl.when(sid == 0)
        def _():
            pltpu.sync_copy(zv_h, den_s)

        pltpu.sync_copy(as_h, as_v)
        pltpu.sync_copy(ad_h, ad_v)
        pltpu.sync_copy(src_h.at[pl.ds(base, per_t)], src_v)
        pltpu.sync_copy(dst_h.at[pl.ds(base, per_t)], dst_v)
        plsc.subcore_barrier()

        neg = jnp.full((16,), -3.4e38, f32)

        def mx(j, carry):
            ms, md = carry
            sl = pl.ds(16 * j, 16)
            return jnp.maximum(ms, as_v[sl]), jnp.maximum(md, ad_v[sl])

        ms, md = lax.fori_loop(0, nvt, mx, (neg, neg))
        m = jnp.max(ms) + jnp.max(md)
        c = jnp.where(m >= 0.0, m, 0.2 * m)
        cvec = jnp.full((16,), c, f32)

        def body(j, _):
            sl = pl.ds(16 * j, 16)
            a = plsc.load_gather(as_v, [src_v[sl]])
            b = plsc.load_gather(ad_v, [dst_v[sl]])
            z = a + b
            z = jnp.where(z >= 0.0, z, 0.2 * z)
            ex_v[sl] = jnp.exp(z - cvec)
            return 0

        lax.fori_loop(0, nv, body, 0)
        pltpu.sync_copy(ex_v, ex_h.at[pl.ds(base, per_t)])
        pltpu.sync_copy(ex_v, den_s.at[dst_v], add=True)

        @pl.when(cid == 0)
        def _():
            e0 = pl.multiple_of(sid * es_str, 8)
            nv_es = jnp.where(sid == 15, 400 // 16, es_str // 16)

            def selfloop(j, _):
                gsl = pl.ds(e0 + 16 * j, 16)
                z = as_v[gsl] + ad_v[gsl]
                z = jnp.where(z >= 0.0, z, 0.2 * z)
                es_v[pl.ds(16 * j, 16)] = jnp.exp(z - cvec)
                return 0

            lax.fori_loop(0, nv_es, selfloop, 0)

            @pl.when(sid < 15)
            def _():
                pltpu.sync_copy(es_v, exs_h.at[pl.ds(e0, es_str)])

            @pl.when(sid == 15)
            def _():
                pltpu.sync_copy(es_v.at[pl.ds(0, 400)],
                                exs_h.at[pl.ds(15 * es_str, 400)])

        plsc.subcore_barrier()

        @pl.when(sid == 0)
        def _():
            pltpu.sync_copy(den_s, den_h.at[cid])

    return k(asrc, adst, src, dst, zvec)


_R = 1000


def _mm1(x, w1):
    def body(x_ref, w_ref, o_ref):
        o_ref[...] = jnp.dot(x_ref[...], w_ref[...], preferred_element_type=f32)

    return pl.pallas_call(
        body,
        grid=(N // _R,),
        in_specs=[
            pl.BlockSpec((_R, D_IN), lambda i: (i, 0)),
            pl.BlockSpec((D_IN, D_H), lambda i: (0, 0)),
        ],
        out_specs=pl.BlockSpec((_R, D_H), lambda i: (i, 0)),
        out_shape=jax.ShapeDtypeStruct((N, D_H), f32),
    )(x, w1)


def _scale_e(e_p, de_p):
    def body(e_ref, d_ref, o_ref):
        de = d_ref[0] + d_ref[1]
        inv = jnp.where(de > 0.0, 1.0 / de, 0.0)
        o_ref[...] = (e_ref[0] + e_ref[1]) * inv

    return pl.pallas_call(
        body,
        grid=(N // _R,),
        in_specs=[
            pl.BlockSpec((NC, _R, D_H), lambda i: (0, i, 0)),
            pl.BlockSpec((NC, _R, 1), lambda i: (0, i, 0)),
        ],
        out_specs=pl.BlockSpec((_R, D_H), lambda i: (i, 0)),
        out_shape=jax.ShapeDtypeStruct((N, D_H), f32),
    )(e_p, de_p)


def _h1_mm2(o_p, dn_p, b1, w2, a_s, a_d):
    def body(o_ref, d_ref, b_ref, w_ref, s_ref, t_ref, xl_ref, as_ref, ad_ref):
        dn = d_ref[0] + d_ref[1]
        inv = jnp.where(dn > 0.0, 1.0 / dn, 0.0)
        h = jnp.maximum((o_ref[0] + o_ref[1]) * inv + b_ref[...], 0.0)
        xl = jnp.dot(h, w_ref[...], preferred_element_type=f32)
        xl_ref[...] = xl
        as_ref[...] = jnp.sum(xl * s_ref[...], axis=-1, keepdims=True)
        ad_ref[...] = jnp.sum(xl * t_ref[...], axis=-1, keepdims=True)

    return pl.pallas_call(
        body,
        grid=(N // _R,),
        in_specs=[
            pl.BlockSpec((NC, _R, D_H), lambda i: (0, i, 0)),
            pl.BlockSpec((NC, _R, 1), lambda i: (0, i, 0)),
            pl.BlockSpec((1, D_H), lambda i: (0, 0)),
            pl.BlockSpec((D_H, D_H), lambda i: (0, 0)),
            pl.BlockSpec((1, D_H), lambda i: (0, 0)),
            pl.BlockSpec((1, D_H), lambda i: (0, 0)),
        ],
        out_specs=[
            pl.BlockSpec((_R, D_H), lambda i: (i, 0)),
            pl.BlockSpec((_R, 1), lambda i: (i, 0)),
            pl.BlockSpec((_R, 1), lambda i: (i, 0)),
        ],
        out_shape=[
            jax.ShapeDtypeStruct((N, D_H), f32),
            jax.ShapeDtypeStruct((N, 1), f32),
            jax.ShapeDtypeStruct((N, 1), f32),
        ],
    )(o_p, dn_p, b1, w2, a_s, a_d)


def _gat_comb(p, exs, xl, den_p, b, w_next=None, a_s=None, a_d=None, d_out=None,
              out_slice=None):
    d = xl.shape[-1]
    fused = w_next is not None
    o = out_slice if out_slice is not None else d

    def body(p_ref, e_ref, x_ref, d_ref, b_ref, *rest):
        if fused:
            w_ref, s_ref, t_ref, xl_ref, as_ref, ad_ref = rest
        else:
            (h_ref,) = rest
        den = d_ref[0] + d_ref[1] + e_ref[...]
        acc = p_ref[0] + p_ref[1] + e_ref[...] * x_ref[...]
        h = jnp.maximum(acc / (den + 1e-16) + b_ref[...], 0.0)
        if fused:
            xl = jnp.dot(h, w_ref[...], preferred_element_type=f32)
            xl_ref[...] = xl
            as_ref[...] = jnp.sum(xl * s_ref[...], axis=-1, keepdims=True)
            ad_ref[...] = jnp.sum(xl * t_ref[...], axis=-1, keepdims=True)
        else:
            h_ref[...] = h[:, :o]

    in_specs = [
        pl.BlockSpec((NC, _R, d), lambda i: (0, i, 0)),
        pl.BlockSpec((_R, 1), lambda i: (i, 0)),
        pl.BlockSpec((_R, d), lambda i: (i, 0)),
        pl.BlockSpec((NC, _R, 1), lambda i: (0, i, 0)),
        pl.BlockSpec((1, d), lambda i: (0, 0)),
    ]
    args = [p, exs, xl, den_p, b]
    if fused:
        in_specs += [
            pl.BlockSpec((d, d_out), lambda i: (0, 0)),
            pl.BlockSpec((1, d_out), lambda i: (0, 0)),
            pl.BlockSpec((1, d_out), lambda i: (0, 0)),
        ]
        args += [w_next, a_s, a_d]
        out_specs = [
            pl.BlockSpec((_R, d_out), lambda i: (i, 0)),
            pl.BlockSpec((_R, 1), lambda i: (i, 0)),
            pl.BlockSpec((_R, 1), lambda i: (i, 0)),
        ]
        out_shape = [
            jax.ShapeDtypeStruct((N, d_out), f32),
            jax.ShapeDtypeStruct((N, 1), f32),
            jax.ShapeDtypeStruct((N, 1), f32),
        ]
    else:
        out_specs = pl.BlockSpec((_R, o), lambda i: (i, 0))
        out_shape = jax.ShapeDtypeStruct((N, o), f32)

    return pl.pallas_call(
        body,
        grid=(N // _R,),
        in_specs=in_specs,
        out_specs=out_specs,
        out_shape=out_shape,
    )(*args)


def kernel(x, edge_index, hyperedge_index, W1, b1, W2, att_src2, att_dst2, b2,
           W3, att_src3, att_dst3, b3):
    nidx = hyperedge_index[0]
    eidx = hyperedge_index[1]
    src = edge_index[0]
    dst = edge_index[1]

    zvec = jnp.zeros((N,), f32)
    z2h = jnp.zeros((N, D_H), f32)
    z2o = jnp.zeros((N, D_OUT), f32)
    ones_t = jnp.ones((NNZ // NW,), f32)

    x1 = _mm1(x, W1)
    dn_p, de_p = _degrees(nidx, eidx, ones_t, zvec)
    e_p = _hop(x1, nidx, eidx, None, z2h, D_H, NNZ, 128)
    e_s = _scale_e(e_p, de_p.reshape(NC, N, 1))
    o_p = _hop(e_s, eidx, nidx, None, z2h, D_H, NNZ, 128)

    xl2, as2, ad2 = _h1_mm2(
        o_p, dn_p.reshape(NC, N, 1), b1.reshape(1, D_H), W2,
        att_src2.reshape(1, D_H), att_dst2.reshape(1, D_H))
    ex2, den2p, exs2 = _gat_alpha(as2.reshape(N), ad2.reshape(N), src, dst, zvec)
    exs2 = exs2.reshape(N, 1)
    p2 = _hop(xl2, src, dst, ex2, z2h, D_H, E, 80)

    pad = D_H - D_OUT
    w3p = jnp.pad(W3, ((0, 0), (0, pad)))
    xl3, as3, ad3 = _gat_comb(
        p2, exs2, xl2, den2p.reshape(NC, N, 1), b2.reshape(1, D_H),
        w_next=w3p, a_s=jnp.pad(att_src3, (0, pad)).reshape(1, D_H),
        a_d=jnp.pad(att_dst3, (0, pad)).reshape(1, D_H), d_out=D_H)
    ex3, den3p, exs3 = _gat_alpha(as3.reshape(N), ad3.reshape(N), src, dst, zvec)
    exs3 = exs3.reshape(N, 1)
    p3 = _hop(xl3, src, dst, ex3, z2h, D_H, E, 80)

    return _gat_comb(p3, exs3, xl3, den3p.reshape(NC, N, 1),
                     jnp.pad(b3, (0, pad)).reshape(1, D_H), out_slice=D_OUT)
